# baseline (speedup 1.0000x reference)
import functools

import jax
import jax.numpy as jnp
from jax import lax
from jax.experimental import pallas as pl
from jax.experimental.pallas import tpu as pltpu
from jax.experimental.pallas import tpu_sc as plsc

_NC, _NS = 2, 16
_NW = _NC * _NS
_L = 16
_NBUF = 8
_AHEAD = 7


def _segsum_sc(y, src3, dst3, with_count):
  N, W = y.shape
  _, NCHUNK, CH = src3.shape
  NP = N + 8
  CZ = 80
  NZCH = N // CZ
  mesh = plsc.VectorSubcoreMesh(core_axis_name="c", subcore_axis_name="s",
                                num_cores=_NC, num_subcores=_NS)
  out_type = [jax.ShapeDtypeStruct((_NC, N, W), jnp.float32)]
  scratch = [
      pltpu.VMEM_SHARED((NP, W), jnp.float32),
      pltpu.VMEM((NCHUNK, CH), jnp.int32),
      pltpu.VMEM((NCHUNK, CH), jnp.int32),
      pltpu.VMEM((CZ, W), jnp.float32),
  ]
  scratch += [pltpu.VMEM((CH, W), jnp.float32) for _ in range(_NBUF)]
  scratch += [pltpu.SemaphoreType.DMA for _ in range(2 * _NBUF)]
  if with_count:
    out_type.append(jax.ShapeDtypeStruct((_NC * N,), jnp.float32))
    scratch += [
        pltpu.VMEM_SHARED((NP,), jnp.float32),
        pltpu.VMEM((CH,), jnp.float32),
        pltpu.VMEM((1008,), jnp.float32),
        pltpu.SemaphoreType.DMA,
    ]

  def body(y_hbm, src_hbm, dst_hbm, *rest):
    if with_count:
      (out_hbm, cnt_hbm, acc_sh, sidx, didx, zbuf, *ring_refs,
       cnt_sh, ones_v, zc, semc) = rest
    else:
      (out_hbm, acc_sh, sidx, didx, zbuf, *ring_refs) = rest
    ring = tuple(zip(ring_refs[:_NBUF], ring_refs[_NBUF:2 * _NBUF],
                     ring_refs[2 * _NBUF:3 * _NBUF]))
    c = lax.axis_index("c")
    s = lax.axis_index("s")
    wid = c * _NS + s
    zero16 = jnp.zeros((_L,), jnp.float32)

    def zrow(i, carry):
      for j in range(W // _L):
        zbuf[i, pl.ds(j * _L, _L)] = zero16
      return carry
    lax.fori_loop(0, CZ, zrow, 0)
    def zacc(k, carry):
      @pl.when(k % _NS == s)
      def _():
        pltpu.sync_copy(zbuf, acc_sh.at[pl.ds(k * CZ, CZ)])
      return carry
    lax.fori_loop(0, NZCH, zacc, 0)

    if with_count:
      def fill_ones(i, carry):
        ones_v[pl.ds(i * _L, _L)] = zero16 + 1.0
        return carry
      lax.fori_loop(0, CH // _L, fill_ones, 0)

      @pl.when(s == 0)
      def _():
        def zfill(i, carry):
          zc[pl.ds(i * _L, _L)] = zero16
          return carry
        lax.fori_loop(0, zc.shape[0] // _L, zfill, 0)
        def zcnt(k, carry):
          pltpu.sync_copy(zc.at[pl.ds(0, 1000)],
                          cnt_sh.at[pl.ds(k * 1000, 1000)])
          return carry
        lax.fori_loop(0, N // 1000, zcnt, 0)

    plsc.subcore_barrier()

    pltpu.sync_copy(src_hbm.at[wid], sidx)
    pltpu.sync_copy(dst_hbm.at[wid], didx)

    LAG = _NBUF - _AHEAD
    for j in range(min(_AHEAD, NCHUNK)):
      pltpu.async_copy(y_hbm.at[sidx.at[j]], ring[j][0], ring[j][1])

    def chunk(i, carry):
      def step(q):
        rows_c, semg_c, sems_c = ring[q]
        rows_n, semg_n, sems_n = ring[(q + _AHEAD) % _NBUF]
        pltpu.make_async_copy(y_hbm.at[sidx.at[i]], rows_c, semg_c).wait()
        pltpu.async_copy(rows_c, acc_sh.at[didx.at[i]], sems_c, add=True)
        if with_count:
          pltpu.async_copy(ones_v, cnt_sh.at[didx.at[i]], semc, add=True)
        @pl.when(i >= LAG)
        def _():
          pltpu.make_async_copy(rows_n, acc_sh.at[didx.at[i]],
                                sems_n).wait()
        @pl.when(i + _AHEAD < NCHUNK)
        def _():
          pltpu.async_copy(y_hbm.at[sidx.at[i + _AHEAD]], rows_n, semg_n)

      for q in range(_NBUF):
        @pl.when(lax.rem(i, _NBUF) == q)
        def _(q=q):
          step(q)
      return carry
    lax.fori_loop(0, NCHUNK, chunk, 0)

    for j in range(max(NCHUNK - LAG, 0), NCHUNK):
      q = j % _NBUF
      pltpu.make_async_copy(ring[q][0], acc_sh.at[didx.at[0]],
                            ring[q][2]).wait()

    if with_count:
      def drainc(i, carry):
        pltpu.make_async_copy(ones_v, cnt_sh.at[didx.at[0]], semc).wait()
        return carry
      lax.fori_loop(0, NCHUNK, drainc, 0)

    plsc.subcore_barrier()

    def cpo(k, carry):
      @pl.when(k % _NS == s)
      def _():
        pltpu.sync_copy(acc_sh.at[pl.ds(k * CZ, CZ)],
                        out_hbm.at[c, pl.ds(k * CZ, CZ)])
      return carry
    lax.fori_loop(0, NZCH, cpo, 0)
    if with_count:
      @pl.when(s == 0)
      def _():
        def cpc(k, carry):
          pltpu.sync_copy(cnt_sh.at[pl.ds(k * 1000, 1000)],
                          cnt_hbm.at[pl.ds(c * N + k * 1000, 1000)])
          return carry
        lax.fori_loop(0, N // 1000, cpc, 0)

  fn = pl.kernel(body, out_type=tuple(out_type), mesh=mesh,
                 scratch_types=tuple(scratch),
                 compiler_params=pltpu.CompilerParams(use_tc_tiling_on_sc=False))
  return fn(y, src3, dst3)


def _tc_in_body(x_ref, wl_ref, wr_ref, bl_ref, y_ref, r_ref):
  x = x_ref[...]
  y_ref[...] = jnp.dot(x, wl_ref[...], preferred_element_type=jnp.float32)
  r_ref[...] = (jnp.dot(x, wr_ref[...], preferred_element_type=jnp.float32)
                + bl_ref[...])


def _tc_in(x, wl, wr, bl):
  N = x.shape[0]
  H = wl.shape[1]
  return pl.pallas_call(
      _tc_in_body,
      out_shape=(jax.ShapeDtypeStruct((N, H), jnp.float32),
                 jax.ShapeDtypeStruct((N, H), jnp.float32)),
  )(x, wl, wr, bl.reshape(1, H))


def _tc_mid_body(p0_ref, p1_ref, c0_ref, c1_ref, r_ref, wl_ref, wr_ref,
                 bl_ref, y_ref, rn_ref):
  cnt = jnp.maximum(c0_ref[...] + c1_ref[...], 1.0)
  h = jnp.maximum((p0_ref[...] + p1_ref[...]) / cnt + r_ref[...], 0.0)
  y_ref[...] = jnp.dot(h, wl_ref[...], preferred_element_type=jnp.float32)
  rn_ref[...] = (jnp.dot(h, wr_ref[...], preferred_element_type=jnp.float32)
                 + bl_ref[...])


def _tc_mid(p0, p1, c0, c1, r, wl, wr, bl):
  N = p0.shape[0]
  H = wl.shape[1]
  return pl.pallas_call(
      _tc_mid_body,
      out_shape=(jax.ShapeDtypeStruct((N, H), jnp.float32),
                 jax.ShapeDtypeStruct((N, H), jnp.float32)),
  )(p0, p1, c0, c1, r, wl, wr, bl.reshape(1, H))


def _tc_fin_body(p0_ref, p1_ref, c0_ref, c1_ref, r_ref, o_ref, *, n_cls):
  cnt = jnp.maximum(c0_ref[...] + c1_ref[...], 1.0)
  z = (p0_ref[...] + p1_ref[...]) / cnt + r_ref[...]
  mask = lax.broadcasted_iota(jnp.int32, z.shape, 1) < n_cls
  zm = jnp.where(mask, z, -1e30)
  m = jnp.max(zm, axis=1, keepdims=True)
  e = jnp.where(mask, jnp.exp(z - m), 0.0)
  ssum = jnp.sum(e, axis=1, keepdims=True)
  o_ref[...] = z - m - jnp.log(ssum)


def _tc_fin(p0, p1, c0, c1, r, n_cls):
  N, W = p0.shape
  return pl.pallas_call(
      functools.partial(_tc_fin_body, n_cls=n_cls),
      out_shape=jax.ShapeDtypeStruct((N, W), jnp.float32),
  )(p0, p1, c0, c1, r)


def kernel(x, edge_index, Wl1, bl1, Wr1, Wl2, bl2, Wr2, Wl3, bl3, Wr3):
  N = x.shape[0]
  E = edge_index.shape[1]
  C = Wl3.shape[1]
  W3 = 16
  CH = 80
  EPT0 = E // _NW
  NCHUNK = (EPT0 + CH - 1) // CH
  PAD = NCHUNK * CH - EPT0

  ei = edge_index.astype(jnp.int32)
  src3 = jnp.pad(ei[0].reshape(_NW, EPT0),
                 ((0, 0), (0, PAD))).reshape(_NW, NCHUNK, CH)
  if PAD:
    padrow = (N + (jnp.arange(_NW, dtype=jnp.int32) % 8))[:, None]
    dst3 = jnp.concatenate(
        [ei[1].reshape(_NW, EPT0),
         jnp.broadcast_to(padrow, (_NW, PAD))], axis=1
    ).reshape(_NW, NCHUNK, CH)
  else:
    dst3 = ei[1].reshape(_NW, NCHUNK, CH)

  y1, r1 = _tc_in(x, Wl1, Wr1, bl1)
  p1, cnt = _segsum_sc(y1, src3, dst3, True)
  cnt = cnt.reshape(_NC, N)
  c0 = cnt[0].reshape(N, 1)
  c1 = cnt[1].reshape(N, 1)

  y2, r2 = _tc_mid(p1[0], p1[1], c0, c1, r1, Wl2, Wr2, bl2)
  p2 = _segsum_sc(y2, src3, dst3, False)[0]

  wl3p = jnp.pad(Wl3, ((0, 0), (0, W3 - C)))
  wr3p = jnp.pad(Wr3, ((0, 0), (0, W3 - C)))
  bl3p = jnp.pad(bl3, (0, W3 - C))
  y3, r3 = _tc_mid(p2[0], p2[1], c0, c1, r2, wl3p, wr3p, bl3p)
  p3 = _segsum_sc(y3, src3, dst3, False)[0]

  out = _tc_fin(p3[0], p3[1], c0, c1, r3, C)
  return out[:, :C]


# speedup vs baseline: 1.1288x; 1.1288x over previous
import functools

import jax
import jax.numpy as jnp
from jax import lax
from jax.experimental import pallas as pl
from jax.experimental.pallas import tpu as pltpu
from jax.experimental.pallas import tpu_sc as plsc

_NC, _NS = 2, 16
_NW = _NC * _NS
_L = 16
_NBUF = 8
_AHEAD = 7


def _segsum_sc(y, src3, dst3, with_count):
  N, W = y.shape
  _, NCHUNK, CH = src3.shape
  NP = N + 8
  CZ = 80
  NZCH = N // CZ
  mesh = plsc.VectorSubcoreMesh(core_axis_name="c", subcore_axis_name="s",
                                num_cores=_NC, num_subcores=_NS)
  out_type = [jax.ShapeDtypeStruct((_NC, N, W), jnp.float32)]
  scratch = [
      pltpu.VMEM_SHARED((NP, W), jnp.float32),
      pltpu.VMEM((NCHUNK, CH), jnp.int32),
      pltpu.VMEM((NCHUNK, CH), jnp.int32),
      pltpu.VMEM((CZ, W), jnp.float32),
  ]
  scratch += [pltpu.VMEM((CH, W), jnp.float32) for _ in range(_NBUF)]
  scratch += [pltpu.SemaphoreType.DMA for _ in range(2 * _NBUF)]
  if with_count:
    out_type.append(jax.ShapeDtypeStruct((_NC * N,), jnp.float32))
    scratch += [
        pltpu.VMEM_SHARED((NP,), jnp.float32),
        pltpu.VMEM((CH,), jnp.float32),
        pltpu.VMEM((1008,), jnp.float32),
        pltpu.SemaphoreType.DMA,
    ]

  def body(y_hbm, src_hbm, dst_hbm, *rest):
    if with_count:
      (out_hbm, cnt_hbm, acc_sh, sidx, didx, zbuf, *ring_refs,
       cnt_sh, ones_v, zc, semc) = rest
    else:
      (out_hbm, acc_sh, sidx, didx, zbuf, *ring_refs) = rest
    ring = tuple(zip(ring_refs[:_NBUF], ring_refs[_NBUF:2 * _NBUF],
                     ring_refs[2 * _NBUF:3 * _NBUF]))
    c = lax.axis_index("c")
    s = lax.axis_index("s")
    wid = c * _NS + s
    zero16 = jnp.zeros((_L,), jnp.float32)

    def zrow(i, carry):
      for j in range(W // _L):
        zbuf[i, pl.ds(j * _L, _L)] = zero16
      return carry
    lax.fori_loop(0, CZ, zrow, 0)
    def zacc(k, carry):
      @pl.when(k % _NS == s)
      def _():
        pltpu.sync_copy(zbuf, acc_sh.at[pl.ds(k * CZ, CZ)])
      return carry
    lax.fori_loop(0, NZCH, zacc, 0)

    if with_count:
      def fill_ones(i, carry):
        ones_v[pl.ds(i * _L, _L)] = zero16 + 1.0
        return carry
      lax.fori_loop(0, CH // _L, fill_ones, 0)

      @pl.when(s == 0)
      def _():
        def zfill(i, carry):
          zc[pl.ds(i * _L, _L)] = zero16
          return carry
        lax.fori_loop(0, zc.shape[0] // _L, zfill, 0)
        def zcnt(k, carry):
          pltpu.sync_copy(zc.at[pl.ds(0, 1000)],
                          cnt_sh.at[pl.ds(k * 1000, 1000)])
          return carry
        lax.fori_loop(0, N // 1000, zcnt, 0)

    plsc.subcore_barrier()

    pltpu.sync_copy(src_hbm.at[wid], sidx)
    pltpu.sync_copy(dst_hbm.at[wid], didx)

    LAG = _NBUF - _AHEAD
    for j in range(min(_AHEAD, NCHUNK)):
      pltpu.async_copy(y_hbm.at[sidx.at[j]], ring[j][0], ring[j][1])

    def chunk(i, carry):
      def step(q):
        rows_c, semg_c, sems_c = ring[q]
        rows_n, semg_n, sems_n = ring[(q + _AHEAD) % _NBUF]
        pltpu.make_async_copy(y_hbm.at[sidx.at[i]], rows_c, semg_c).wait()
        pltpu.async_copy(rows_c, acc_sh.at[didx.at[i]], sems_c, add=True)
        if with_count:
          pltpu.async_copy(ones_v, cnt_sh.at[didx.at[i]], semc, add=True)
        @pl.when(i >= LAG)
        def _():
          pltpu.make_async_copy(rows_n, acc_sh.at[didx.at[i]],
                                sems_n).wait()
        @pl.when(i + _AHEAD < NCHUNK)
        def _():
          pltpu.async_copy(y_hbm.at[sidx.at[i + _AHEAD]], rows_n, semg_n)

      for q in range(_NBUF):
        @pl.when(lax.rem(i, _NBUF) == q)
        def _(q=q):
          step(q)
      return carry
    lax.fori_loop(0, NCHUNK, chunk, 0)

    for j in range(max(NCHUNK - LAG, 0), NCHUNK):
      q = j % _NBUF
      pltpu.make_async_copy(ring[q][0], acc_sh.at[didx.at[0]],
                            ring[q][2]).wait()

    if with_count:
      def drainc(i, carry):
        pltpu.make_async_copy(ones_v, cnt_sh.at[didx.at[0]], semc).wait()
        return carry
      lax.fori_loop(0, NCHUNK, drainc, 0)

    plsc.subcore_barrier()

    def cpo(k, carry):
      @pl.when(k % _NS == s)
      def _():
        pltpu.sync_copy(acc_sh.at[pl.ds(k * CZ, CZ)],
                        out_hbm.at[c, pl.ds(k * CZ, CZ)])
      return carry
    lax.fori_loop(0, NZCH, cpo, 0)
    if with_count:
      @pl.when(s == 0)
      def _():
        def cpc(k, carry):
          pltpu.sync_copy(cnt_sh.at[pl.ds(k * 1000, 1000)],
                          cnt_hbm.at[pl.ds(c * N + k * 1000, 1000)])
          return carry
        lax.fori_loop(0, N // 1000, cpc, 0)

  fn = pl.kernel(body, out_type=tuple(out_type), mesh=mesh,
                 scratch_types=tuple(scratch),
                 compiler_params=pltpu.CompilerParams(use_tc_tiling_on_sc=False))
  return fn(y, src3, dst3)


def _tc_in_body(x_ref, wl_ref, wr_ref, bl_ref, y_ref, r_ref):
  x = x_ref[...]
  y_ref[...] = jnp.dot(x, wl_ref[...], preferred_element_type=jnp.float32)
  r_ref[...] = (jnp.dot(x, wr_ref[...], preferred_element_type=jnp.float32)
                + bl_ref[...])


def _tc_in(x, wl, wr, bl):
  N = x.shape[0]
  H = wl.shape[1]
  return pl.pallas_call(
      _tc_in_body,
      out_shape=(jax.ShapeDtypeStruct((N, H), jnp.float32),
                 jax.ShapeDtypeStruct((N, H), jnp.float32)),
  )(x, wl, wr, bl.reshape(1, H))


def _recip_cnt(cnt_ref, n):
  c = jnp.maximum(cnt_ref[pl.ds(0, n)] + cnt_ref[pl.ds(n, n)], 1.0)
  return (1.0 / c).reshape(n, 1)


def _tc_mid_body(p_ref, cnt_ref, r_ref, wl_ref, wr_ref, bl_ref, y_ref,
                 rn_ref):
  n = r_ref.shape[0]
  rec = _recip_cnt(cnt_ref, n)
  h = jnp.maximum((p_ref[0] + p_ref[1]) * rec + r_ref[...], 0.0)
  y_ref[...] = jnp.dot(h, wl_ref[...], preferred_element_type=jnp.float32)
  rn_ref[...] = (jnp.dot(h, wr_ref[...], preferred_element_type=jnp.float32)
                 + bl_ref[...])


def _tc_mid(p, cnt, r, wl, wr, bl):
  N = r.shape[0]
  H = wl.shape[1]
  return pl.pallas_call(
      _tc_mid_body,
      out_shape=(jax.ShapeDtypeStruct((N, H), jnp.float32),
                 jax.ShapeDtypeStruct((N, H), jnp.float32)),
  )(p, cnt, r, wl, wr, bl.reshape(1, H))


def _tc_fin_body(p_ref, cnt_ref, r_ref, o_ref, *, n_cls):
  n = r_ref.shape[0]
  rec = _recip_cnt(cnt_ref, n)
  z = (p_ref[0] + p_ref[1]) * rec + r_ref[...]
  mask = lax.broadcasted_iota(jnp.int32, z.shape, 1) < n_cls
  zm = jnp.where(mask, z, -1e30)
  m = jnp.max(zm, axis=1, keepdims=True)
  e = jnp.where(mask, jnp.exp(z - m), 0.0)
  ssum = jnp.sum(e, axis=1, keepdims=True)
  o_ref[...] = z - m - jnp.log(ssum)


def _tc_fin(p, cnt, r, n_cls):
  N, W = r.shape
  return pl.pallas_call(
      functools.partial(_tc_fin_body, n_cls=n_cls),
      out_shape=jax.ShapeDtypeStruct((N, W), jnp.float32),
  )(p, cnt, r)


def kernel(x, edge_index, Wl1, bl1, Wr1, Wl2, bl2, Wr2, Wl3, bl3, Wr3):
  N = x.shape[0]
  E = edge_index.shape[1]
  C = Wl3.shape[1]
  W3 = 16
  CH = 80
  EPT0 = E // _NW
  NCHUNK = (EPT0 + CH - 1) // CH
  PAD = NCHUNK * CH - EPT0

  ei = edge_index.astype(jnp.int32)
  src3 = jnp.pad(ei[0].reshape(_NW, EPT0),
                 ((0, 0), (0, PAD))).reshape(_NW, NCHUNK, CH)
  if PAD:
    padrow = (N + (jnp.arange(_NW, dtype=jnp.int32) % 8))[:, None]
    dst3 = jnp.concatenate(
        [ei[1].reshape(_NW, EPT0),
         jnp.broadcast_to(padrow, (_NW, PAD))], axis=1
    ).reshape(_NW, NCHUNK, CH)
  else:
    dst3 = ei[1].reshape(_NW, NCHUNK, CH)

  y1, r1 = _tc_in(x, Wl1, Wr1, bl1)
  p1, cnt = _segsum_sc(y1, src3, dst3, True)

  y2, r2 = _tc_mid(p1, cnt, r1, Wl2, Wr2, bl2)
  p2 = _segsum_sc(y2, src3, dst3, False)[0]

  wl3p = jnp.pad(Wl3, ((0, 0), (0, W3 - C)))
  wr3p = jnp.pad(Wr3, ((0, 0), (0, W3 - C)))
  bl3p = jnp.pad(bl3, (0, W3 - C))
  y3, r3 = _tc_mid(p2, cnt, r2, wl3p, wr3p, bl3p)
  p3 = _segsum_sc(y3, src3, dst3, False)[0]

  out = _tc_fin(p3, cnt, r3, C)
  return out[:, :C]
